# baseline (speedup 1.0000x reference)
import functools

import jax
import jax.numpy as jnp
from jax import lax
from jax.experimental import pallas as pl
from jax.experimental.pallas import tpu as pltpu
from jax.experimental.pallas import tpu_sc as plsc

N = 10000
D = 128
NC = 2
NS = 16
NW = NC * NS
CHUNK = 128
N0_CHUNKS = 105
N1_CHUNKS = 55
ACC_N = 10240
PAD_DST = N


def _mesh():
    return plsc.VectorSubcoreMesh(
        core_axis_name="c", subcore_axis_name="s", num_cores=NC, num_subcores=NS
    )


@functools.lru_cache(maxsize=None)
def _deg_kernel(nmax):
    @functools.partial(
        pl.kernel,
        out_type=jax.ShapeDtypeStruct((NC, ACC_N), jnp.float32),
        mesh=_mesh(),
        scratch_types=[
            pltpu.VMEM((nmax, CHUNK), jnp.int32),
            pltpu.VMEM((CHUNK,), jnp.float32),
            pltpu.VMEM((ACC_N // NS,), jnp.float32),
            pltpu.VMEM_SHARED((ACC_N,), jnp.float32),
        ],
    )
    def deg_k(dsts_hbm, ones_hbm, zeros_hbm, out_hbm, dst_v, ones_v, z_v, acc_sh):
        cid = lax.axis_index("c")
        sid = lax.axis_index("s")
        wid = sid * NC + cid
        nch = jnp.where(cid == 0, N0_CHUNKS, N1_CHUNKS)
        seg = ACC_N // NS
        pltpu.sync_copy(dsts_hbm.at[wid], dst_v)
        pltpu.sync_copy(ones_hbm, ones_v)
        pltpu.sync_copy(zeros_hbm, z_v)
        pltpu.sync_copy(z_v, acc_sh.at[pl.ds(sid * seg, seg)])
        plsc.subcore_barrier()

        def body(j, c):
            pltpu.sync_copy(ones_v, acc_sh.at[dst_v.at[j]], add=True)
            return c

        lax.fori_loop(0, nch, body, 0)
        plsc.subcore_barrier()
        pltpu.sync_copy(
            acc_sh.at[pl.ds(sid * seg, seg)], out_hbm.at[cid, pl.ds(sid * seg, seg)]
        )

    return deg_k


@functools.lru_cache(maxsize=None)
def _agg_kernel(nmax):
    @functools.partial(
        pl.kernel,
        out_type=jax.ShapeDtypeStruct((NC, ACC_N, D), jnp.float32),
        mesh=_mesh(),
        scratch_types=[
            pltpu.VMEM((nmax, CHUNK), jnp.int32),
            pltpu.VMEM((nmax, CHUNK), jnp.int32),
            pltpu.VMEM((CHUNK, D), jnp.float32),
            pltpu.VMEM_SHARED((ACC_N, D), jnp.float32),
            pltpu.SemaphoreType.DMA,
        ],
    )
    def agg_k(table_hbm, srcs_hbm, dsts_hbm, zacc_hbm, out_hbm,
              src_v, dst_v, rows_v, acc_sh, sem):
        cid = lax.axis_index("c")
        sid = lax.axis_index("s")
        wid = sid * NC + cid
        nch = jnp.where(cid == 0, N0_CHUNKS, N1_CHUNKS)
        seg = ACC_N // NS
        with jax.named_scope("agg_setup"):
            pltpu.sync_copy(srcs_hbm.at[wid], src_v)
            pltpu.sync_copy(dsts_hbm.at[wid], dst_v)
            pltpu.sync_copy(zacc_hbm, acc_sh.at[pl.ds(sid * seg, seg)])
            plsc.subcore_barrier()

        def body(j, c):
            with jax.named_scope("agg_gather"):
                pltpu.async_copy(table_hbm.at[src_v.at[j]], rows_v, sem).wait()
            with jax.named_scope("agg_scatter"):
                pltpu.sync_copy(rows_v, acc_sh.at[dst_v.at[j]], add=True)
            return c

        with jax.named_scope("agg_loop"):
            lax.fori_loop(0, nch, body, 0)
            plsc.subcore_barrier()
        with jax.named_scope("agg_writeback"):
            pltpu.sync_copy(
                acc_sh.at[pl.ds(sid * seg, seg)],
                out_hbm.at[cid, pl.ds(sid * seg, seg)],
            )

    return agg_k


def _tc1(x, w1, d0, d1):
    def body(x_ref, w_ref, d0_ref, d1_ref, h_ref, dinv_ref):
        dinv = lax.rsqrt(d0_ref[...] + d1_ref[...] + 1.0)
        h = jnp.dot(x_ref[...], w_ref[...], preferred_element_type=jnp.float32)
        h_ref[...] = h * dinv
        dinv_ref[...] = dinv

    return pl.pallas_call(
        body,
        out_shape=(
            jax.ShapeDtypeStruct((N, D), jnp.float32),
            jax.ShapeDtypeStruct((N, 1), jnp.float32),
        ),
    )(x, w1, d0, d1)


def _tc2(a0, a1, h1, dinv, b1, w2):
    def body(a0_ref, a1_ref, h1_ref, dinv_ref, b_ref, w_ref, h2_ref):
        z = dinv_ref[...] * (a0_ref[...] + a1_ref[...] + h1_ref[...]) + b_ref[...]
        z = jnp.maximum(z, 0.0)
        h2_ref[...] = (
            jnp.dot(z, w_ref[...], preferred_element_type=jnp.float32) * dinv_ref[...]
        )

    return pl.pallas_call(
        body, out_shape=jax.ShapeDtypeStruct((N, D), jnp.float32)
    )(a0, a1, h1, dinv, b1, w2)


def _tc3(a0, a1, h2, dinv, b2):
    def body(a0_ref, a1_ref, h2_ref, dinv_ref, b_ref, o_ref):
        o_ref[...] = (
            dinv_ref[...] * (a0_ref[...] + a1_ref[...] + h2_ref[...]) + b_ref[...]
        )

    return pl.pallas_call(
        body, out_shape=jax.ShapeDtypeStruct((N, D), jnp.float32)
    )(a0, a1, h2, dinv, b2)


def kernel(x, edge_index, W1, b1, W2, b2):
    E = edge_index.shape[1]
    tot_chunks = NS * (N0_CHUNKS + N1_CHUNKS)
    assert tot_chunks * CHUNK >= E
    pad = tot_chunks * CHUNK - E

    def slabs(flat, fill):
        ch = jnp.concatenate(
            [flat, jnp.full((pad,), fill, jnp.int32)]
        ).reshape(tot_chunks, CHUNK)
        s0 = ch[: NS * N0_CHUNKS].reshape(NS, 1, N0_CHUNKS, CHUNK)
        s1 = ch[NS * N0_CHUNKS :].reshape(NS, N1_CHUNKS, CHUNK)
        s1 = jnp.pad(
            s1, ((0, 0), (0, N0_CHUNKS - N1_CHUNKS), (0, 0)), constant_values=fill
        ).reshape(NS, 1, N0_CHUNKS, CHUNK)
        return jnp.concatenate([s0, s1], axis=1).reshape(NW, N0_CHUNKS, CHUNK)

    src = slabs(edge_index[0], 0)
    dst = slabs(edge_index[1], PAD_DST)
    ones = jnp.ones((CHUNK,), jnp.float32)
    zseg = jnp.zeros((ACC_N // NS,), jnp.float32)
    zacc = jnp.zeros((ACC_N // NS, D), jnp.float32)

    deg_parts = _deg_kernel(N0_CHUNKS)(dst, ones, zseg)
    d0 = deg_parts[0, :N][:, None]
    d1 = deg_parts[1, :N][:, None]
    h1, dinv = _tc1(x, W1, d0, d1)

    agg = _agg_kernel(N0_CHUNKS)
    a = agg(h1, src, dst, zacc)
    h2 = _tc2(a[0, :N], a[1, :N], h1, dinv, jnp.reshape(b1, (1, D)), W2)
    a2 = agg(h2, src, dst, zacc)
    return _tc3(a2[0, :N], a2[1, :N], h2, dinv, jnp.reshape(b2, (1, D)))


# speedup vs baseline: 2.4752x; 2.4752x over previous
import functools

import jax
import jax.numpy as jnp
from jax import lax
from jax.experimental import pallas as pl
from jax.experimental.pallas import tpu as pltpu
from jax.experimental.pallas import tpu_sc as plsc

N = 10000
D = 128
NC = 2
NS = 16
NW = NC * NS
CHUNK = 128
N0_CHUNKS = 80
N1_CHUNKS = 80
ACC_N = 10240
PAD_DST = N


def _mesh():
    return plsc.VectorSubcoreMesh(
        core_axis_name="c", subcore_axis_name="s", num_cores=NC, num_subcores=NS
    )


@functools.lru_cache(maxsize=None)
def _deg_kernel(nmax):
    @functools.partial(
        pl.kernel,
        out_type=jax.ShapeDtypeStruct((NC, ACC_N), jnp.float32),
        mesh=_mesh(),
        scratch_types=[
            pltpu.VMEM((nmax, CHUNK), jnp.int32),
            pltpu.VMEM((CHUNK,), jnp.float32),
            pltpu.VMEM((ACC_N // NS,), jnp.float32),
            pltpu.VMEM_SHARED((ACC_N,), jnp.float32),
        ],
    )
    def deg_k(dsts_hbm, ones_hbm, zeros_hbm, out_hbm, dst_v, ones_v, z_v, acc_sh):
        cid = lax.axis_index("c")
        sid = lax.axis_index("s")
        wid = sid * NC + cid
        nch = jnp.where(cid == 0, N0_CHUNKS, N1_CHUNKS)
        seg = ACC_N // NS
        pltpu.sync_copy(dsts_hbm.at[wid], dst_v)
        pltpu.sync_copy(ones_hbm, ones_v)
        pltpu.sync_copy(zeros_hbm, z_v)
        pltpu.sync_copy(z_v, acc_sh.at[pl.ds(sid * seg, seg)])
        plsc.subcore_barrier()

        def body(j, c):
            pltpu.sync_copy(ones_v, acc_sh.at[dst_v.at[j]], add=True)
            return c

        lax.fori_loop(0, nch, body, 0)
        plsc.subcore_barrier()
        pltpu.sync_copy(
            acc_sh.at[pl.ds(sid * seg, seg)], out_hbm.at[cid, pl.ds(sid * seg, seg)]
        )

    return deg_k


@functools.lru_cache(maxsize=None)
def _agg_kernel(nmax):
    @functools.partial(
        pl.kernel,
        out_type=jax.ShapeDtypeStruct((NC, ACC_N, D), jnp.float32),
        mesh=_mesh(),
        scratch_types=[
            pltpu.VMEM((nmax, CHUNK), jnp.int32),
            pltpu.VMEM((nmax, CHUNK), jnp.int32),
            pltpu.VMEM((CHUNK, D), jnp.float32),
            pltpu.VMEM_SHARED((ACC_N, D), jnp.float32),
            pltpu.SemaphoreType.DMA,
        ],
    )
    def agg_k(table_hbm, srcs_hbm, dsts_hbm, zacc_hbm, out_hbm,
              src_v, dst_v, rows_v, acc_sh, sem):
        cid = lax.axis_index("c")
        sid = lax.axis_index("s")
        wid = sid * NC + cid
        nch = jnp.where(cid == 0, N0_CHUNKS, N1_CHUNKS)
        seg = ACC_N // NS
        with jax.named_scope("agg_setup"):
            pltpu.sync_copy(srcs_hbm.at[wid], src_v)
            pltpu.sync_copy(dsts_hbm.at[wid], dst_v)
            pltpu.sync_copy(zacc_hbm, acc_sh.at[pl.ds(sid * seg, seg)])
            plsc.subcore_barrier()

        def body(j, c):
            with jax.named_scope("agg_gather"):
                pltpu.async_copy(table_hbm.at[src_v.at[j]], rows_v, sem).wait()
            with jax.named_scope("agg_scatter"):
                pltpu.sync_copy(rows_v, acc_sh.at[dst_v.at[j]], add=True)
            return c

        with jax.named_scope("agg_loop"):
            lax.fori_loop(0, nch, body, 0)
            plsc.subcore_barrier()
        with jax.named_scope("agg_writeback"):
            pltpu.sync_copy(
                acc_sh.at[pl.ds(sid * seg, seg)],
                out_hbm.at[cid, pl.ds(sid * seg, seg)],
            )

    return agg_k


def _tc1(x, w1, d0, d1):
    def body(x_ref, w_ref, d0_ref, d1_ref, h_ref, dinv_ref):
        dinv = lax.rsqrt(d0_ref[...] + d1_ref[...] + 1.0)
        h = jnp.dot(x_ref[...], w_ref[...], preferred_element_type=jnp.float32)
        h_ref[...] = h * dinv
        dinv_ref[...] = dinv

    return pl.pallas_call(
        body,
        out_shape=(
            jax.ShapeDtypeStruct((N, D), jnp.float32),
            jax.ShapeDtypeStruct((N, 1), jnp.float32),
        ),
    )(x, w1, d0, d1)


def _tc2(a0, a1, h1, dinv, b1, w2):
    def body(a0_ref, a1_ref, h1_ref, dinv_ref, b_ref, w_ref, h2_ref):
        z = dinv_ref[...] * (a0_ref[...] + a1_ref[...] + h1_ref[...]) + b_ref[...]
        z = jnp.maximum(z, 0.0)
        h2_ref[...] = (
            jnp.dot(z, w_ref[...], preferred_element_type=jnp.float32) * dinv_ref[...]
        )

    return pl.pallas_call(
        body, out_shape=jax.ShapeDtypeStruct((N, D), jnp.float32)
    )(a0, a1, h1, dinv, b1, w2)


def _tc3(a0, a1, h2, dinv, b2):
    def body(a0_ref, a1_ref, h2_ref, dinv_ref, b_ref, o_ref):
        o_ref[...] = (
            dinv_ref[...] * (a0_ref[...] + a1_ref[...] + h2_ref[...]) + b_ref[...]
        )

    return pl.pallas_call(
        body, out_shape=jax.ShapeDtypeStruct((N, D), jnp.float32)
    )(a0, a1, h2, dinv, b2)


def kernel(x, edge_index, W1, b1, W2, b2):
    E = edge_index.shape[1]
    tot_chunks = NS * (N0_CHUNKS + N1_CHUNKS)
    assert tot_chunks * CHUNK >= E
    pad = tot_chunks * CHUNK - E

    def slabs(flat, fill_arr):
        ch = jnp.concatenate([flat, fill_arr]).reshape(tot_chunks, CHUNK)
        s0 = ch[: NS * N0_CHUNKS].reshape(NS, 1, N0_CHUNKS, CHUNK)
        s1 = ch[NS * N0_CHUNKS :].reshape(NS, 1, N1_CHUNKS, CHUNK)
        if N0_CHUNKS != N1_CHUNKS:
            raise ValueError("uneven split needs chunk-dim padding")
        return jnp.concatenate([s0, s1], axis=1).reshape(NW, N0_CHUNKS, CHUNK)

    pad_src = jnp.arange(pad, dtype=jnp.int32) % N
    pad_dst = N + (jnp.arange(pad, dtype=jnp.int32) % (ACC_N - N))
    src = slabs(edge_index[0], pad_src)
    dst = slabs(edge_index[1], pad_dst)
    ones = jnp.ones((CHUNK,), jnp.float32)
    zseg = jnp.zeros((ACC_N // NS,), jnp.float32)
    zacc = jnp.zeros((ACC_N // NS, D), jnp.float32)

    deg_parts = _deg_kernel(N0_CHUNKS)(dst, ones, zseg)
    d0 = deg_parts[0, :N][:, None]
    d1 = deg_parts[1, :N][:, None]
    h1, dinv = _tc1(x, W1, d0, d1)

    agg = _agg_kernel(N0_CHUNKS)
    a = agg(h1, src, dst, zacc)
    h2 = _tc2(a[0, :N], a[1, :N], h1, dinv, jnp.reshape(b1, (1, D)), W2)
    a2 = agg(h2, src, dst, zacc)
    return _tc3(a2[0, :N], a2[1, :N], h2, dinv, jnp.reshape(b2, (1, D)))


# speedup vs baseline: 3.3906x; 1.3698x over previous
import functools

import jax
import jax.numpy as jnp
from jax import lax
from jax.experimental import pallas as pl
from jax.experimental.pallas import tpu as pltpu
from jax.experimental.pallas import tpu_sc as plsc

N = 10000
D = 128
NC = 2
NS = 16
NW = NC * NS
CHUNK = 128
N0_CHUNKS = 80
N1_CHUNKS = 80
WIN = 16
ACC_N = 10240
PAD_DST = N


def _mesh():
    return plsc.VectorSubcoreMesh(
        core_axis_name="c", subcore_axis_name="s", num_cores=NC, num_subcores=NS
    )


@functools.lru_cache(maxsize=None)
def _deg_kernel(nmax):
    @functools.partial(
        pl.kernel,
        out_type=jax.ShapeDtypeStruct((NC, ACC_N), jnp.float32),
        mesh=_mesh(),
        scratch_types=[
            pltpu.VMEM((nmax, CHUNK), jnp.int32),
            pltpu.VMEM((CHUNK,), jnp.float32),
            pltpu.VMEM((ACC_N // NS,), jnp.float32),
            pltpu.VMEM_SHARED((ACC_N,), jnp.float32),
        ],
    )
    def deg_k(dsts_hbm, ones_hbm, zeros_hbm, out_hbm, dst_v, ones_v, z_v, acc_sh):
        cid = lax.axis_index("c")
        sid = lax.axis_index("s")
        wid = sid * NC + cid
        nch = jnp.where(cid == 0, N0_CHUNKS, N1_CHUNKS)
        seg = ACC_N // NS
        pltpu.sync_copy(dsts_hbm.at[wid], dst_v)
        pltpu.sync_copy(ones_hbm, ones_v)
        pltpu.sync_copy(zeros_hbm, z_v)
        pltpu.sync_copy(z_v, acc_sh.at[pl.ds(sid * seg, seg)])
        plsc.subcore_barrier()

        def body(j, c):
            pltpu.sync_copy(ones_v, acc_sh.at[dst_v.at[j]], add=True)
            return c

        lax.fori_loop(0, nch, body, 0)
        plsc.subcore_barrier()
        pltpu.sync_copy(
            acc_sh.at[pl.ds(sid * seg, seg)], out_hbm.at[cid, pl.ds(sid * seg, seg)]
        )

    return deg_k


@functools.lru_cache(maxsize=None)
def _agg_kernel(nmax):
    @functools.partial(
        pl.kernel,
        out_type=jax.ShapeDtypeStruct((NC, ACC_N, D), jnp.float32),
        mesh=_mesh(),
        scratch_types=[
            pltpu.VMEM((WIN, CHUNK), jnp.int32),
            pltpu.VMEM((WIN, CHUNK), jnp.int32),
            pltpu.VMEM((2, CHUNK, D), jnp.float32),
            pltpu.VMEM_SHARED((ACC_N, D), jnp.float32),
            pltpu.SemaphoreType.DMA,
        ],
    )
    def agg_k(table_hbm, srcs_hbm, dsts_hbm, zacc_hbm, out_hbm,
              src_v, dst_v, rows_v, acc_sh, sem):
        cid = lax.axis_index("c")
        sid = lax.axis_index("s")
        wid = sid * NC + cid
        seg = ACC_N // NS
        with jax.named_scope("agg_setup"):
            pltpu.sync_copy(zacc_hbm, acc_sh.at[pl.ds(sid * seg, seg)])
            plsc.subcore_barrier()

        def win_body(w, cw):
            pltpu.sync_copy(srcs_hbm.at[wid, pl.ds(w * WIN, WIN)], src_v)
            pltpu.sync_copy(dsts_hbm.at[wid, pl.ds(w * WIN, WIN)], dst_v)

            def body(j, c):
                @pl.when(j < WIN)
                def _():
                    pltpu.async_copy(
                        table_hbm.at[src_v.at[j]], rows_v.at[j % 2], sem
                    )

                @pl.when(j > 0)
                def _():
                    jm = j - 1
                    pltpu.make_async_copy(
                        table_hbm.at[src_v.at[jm]], rows_v.at[jm % 2], sem
                    ).wait()
                    pltpu.sync_copy(
                        rows_v.at[jm % 2], acc_sh.at[dst_v.at[jm]], add=True
                    )

                return c

            lax.fori_loop(0, WIN + 1, body, 0)
            return cw

        with jax.named_scope("agg_loop"):
            lax.fori_loop(0, N0_CHUNKS // WIN, win_body, 0)
            plsc.subcore_barrier()
        with jax.named_scope("agg_writeback"):
            pltpu.sync_copy(
                acc_sh.at[pl.ds(sid * seg, seg)],
                out_hbm.at[cid, pl.ds(sid * seg, seg)],
            )

    return agg_k


def _tc1(x, w1, d0, d1):
    def body(x_ref, w_ref, d0_ref, d1_ref, h_ref, dinv_ref):
        dinv = lax.rsqrt(d0_ref[...] + d1_ref[...] + 1.0)
        h = jnp.dot(x_ref[...], w_ref[...], preferred_element_type=jnp.float32)
        h_ref[...] = h * dinv
        dinv_ref[...] = dinv

    return pl.pallas_call(
        body,
        out_shape=(
            jax.ShapeDtypeStruct((N, D), jnp.float32),
            jax.ShapeDtypeStruct((N, 1), jnp.float32),
        ),
    )(x, w1, d0, d1)


def _tc2(a0, a1, h1, dinv, b1, w2):
    def body(a0_ref, a1_ref, h1_ref, dinv_ref, b_ref, w_ref, h2_ref):
        z = dinv_ref[...] * (a0_ref[...] + a1_ref[...] + h1_ref[...]) + b_ref[...]
        z = jnp.maximum(z, 0.0)
        h2_ref[...] = (
            jnp.dot(z, w_ref[...], preferred_element_type=jnp.float32) * dinv_ref[...]
        )

    return pl.pallas_call(
        body, out_shape=jax.ShapeDtypeStruct((N, D), jnp.float32)
    )(a0, a1, h1, dinv, b1, w2)


def _tc3(a0, a1, h2, dinv, b2):
    def body(a0_ref, a1_ref, h2_ref, dinv_ref, b_ref, o_ref):
        o_ref[...] = (
            dinv_ref[...] * (a0_ref[...] + a1_ref[...] + h2_ref[...]) + b_ref[...]
        )

    return pl.pallas_call(
        body, out_shape=jax.ShapeDtypeStruct((N, D), jnp.float32)
    )(a0, a1, h2, dinv, b2)


def kernel(x, edge_index, W1, b1, W2, b2):
    E = edge_index.shape[1]
    tot_chunks = NS * (N0_CHUNKS + N1_CHUNKS)
    assert tot_chunks * CHUNK >= E
    pad = tot_chunks * CHUNK - E

    def slabs(flat, fill_arr):
        ch = jnp.concatenate([flat, fill_arr]).reshape(tot_chunks, CHUNK)
        s0 = ch[: NS * N0_CHUNKS].reshape(NS, 1, N0_CHUNKS, CHUNK)
        s1 = ch[NS * N0_CHUNKS :].reshape(NS, 1, N1_CHUNKS, CHUNK)
        if N0_CHUNKS != N1_CHUNKS:
            raise ValueError("uneven split needs chunk-dim padding")
        return jnp.concatenate([s0, s1], axis=1).reshape(NW, N0_CHUNKS, CHUNK)

    pad_src = jnp.arange(pad, dtype=jnp.int32) % N
    pad_dst = N + (jnp.arange(pad, dtype=jnp.int32) % (ACC_N - N))
    src = slabs(edge_index[0], pad_src)
    dst = slabs(edge_index[1], pad_dst)
    ones = jnp.ones((CHUNK,), jnp.float32)
    zseg = jnp.zeros((ACC_N // NS,), jnp.float32)
    zacc = jnp.zeros((ACC_N // NS, D), jnp.float32)

    deg_parts = _deg_kernel(N0_CHUNKS)(dst, ones, zseg)
    d0 = deg_parts[0, :N][:, None]
    d1 = deg_parts[1, :N][:, None]
    h1, dinv = _tc1(x, W1, d0, d1)

    agg = _agg_kernel(N0_CHUNKS)
    a = agg(h1, src, dst, zacc)
    h2 = _tc2(a[0, :N], a[1, :N], h1, dinv, jnp.reshape(b1, (1, D)), W2)
    a2 = agg(h2, src, dst, zacc)
    return _tc3(a2[0, :N], a2[1, :N], h2, dinv, jnp.reshape(b2, (1, D)))
